# baseline (speedup 1.0000x reference)
import jax
import jax.numpy as jnp
from jax.experimental import pallas as pl
from jax.experimental.pallas import tpu as pltpu

_NDMA = 16


def _copy_body(x_ref, o_ref, sems):
    n = x_ref.shape[0] // _NDMA
    for i in range(_NDMA):
        pltpu.make_async_copy(
            x_ref.at[pl.ds(i * n, n)], o_ref.at[pl.ds(i * n, n)], sems.at[i]
        ).start()
    for i in range(_NDMA):
        pltpu.make_async_copy(
            x_ref.at[pl.ds(i * n, n)], o_ref.at[pl.ds(i * n, n)], sems.at[i]
        ).wait()


def kernel(images_batch, masks_batch):
    del masks_batch
    B, ve_dim, feature_dim = images_batch.shape
    rows = B * ve_dim
    flat = images_batch.reshape(rows, feature_dim)
    out = pl.pallas_call(
        _copy_body,
        in_specs=[pl.BlockSpec(memory_space=pl.ANY)],
        out_specs=pl.BlockSpec(memory_space=pl.ANY),
        scratch_shapes=[pltpu.SemaphoreType.DMA((_NDMA,))],
        out_shape=jax.ShapeDtypeStruct((rows, feature_dim), flat.dtype),
    )(flat)
    return out.reshape(B, ve_dim, feature_dim)


# speedup vs baseline: 48.2661x; 48.2661x over previous
import jax
import jax.numpy as jnp
from jax.experimental import pallas as pl
from jax.experimental.pallas import tpu as pltpu

_CHUNK_ROWS = 1024
_NBUF = 4


def _copy_body(x_ref, o_ref, *bufs_and_sems):
    bufs = bufs_and_sems[:_NBUF]
    gsems = bufs_and_sems[_NBUF]
    ssems = bufs_and_sems[_NBUF + 1]
    nchunks = x_ref.shape[0] // _CHUNK_ROWS

    def gather(i):
        b = i % _NBUF
        return pltpu.make_async_copy(
            x_ref.at[pl.ds(i * _CHUNK_ROWS, _CHUNK_ROWS)], bufs[b], gsems.at[b])

    def scatter(i):
        b = i % _NBUF
        return pltpu.make_async_copy(
            bufs[b], o_ref.at[pl.ds(i * _CHUNK_ROWS, _CHUNK_ROWS)], ssems.at[b])

    for i in range(_NBUF - 1):
        gather(i).start()
    for i in range(nchunks):
        gather(i).wait()
        scatter(i).start()
        nxt = i + _NBUF - 1
        if nxt < nchunks:
            if nxt - _NBUF >= 0:
                scatter(nxt - _NBUF).wait()
            gather(nxt).start()
    for i in range(max(0, nchunks - _NBUF), nchunks):
        scatter(i).wait()


def kernel(images_batch, masks_batch):
    del masks_batch
    B, ve_dim, feature_dim = images_batch.shape
    rows = B * ve_dim
    flat = images_batch.reshape(rows, feature_dim)
    out = pl.pallas_call(
        _copy_body,
        in_specs=[pl.BlockSpec(memory_space=pl.ANY)],
        out_specs=pl.BlockSpec(memory_space=pl.ANY),
        scratch_shapes=(
            [pltpu.VMEM((_CHUNK_ROWS, feature_dim), jnp.float32)
             for _ in range(_NBUF)]
            + [pltpu.SemaphoreType.DMA((_NBUF,)),
               pltpu.SemaphoreType.DMA((_NBUF,))]
        ),
        out_shape=jax.ShapeDtypeStruct((rows, feature_dim), flat.dtype),
    )(flat)
    return out.reshape(B, ve_dim, feature_dim)


# speedup vs baseline: 48.5170x; 1.0052x over previous
import jax
import jax.numpy as jnp
from jax.experimental import pallas as pl
from jax.experimental.pallas import tpu as pltpu

_CHUNK_ROWS = 2048
_NBUF = 4


def _copy_body(x_ref, o_ref, *bufs_and_sems):
    bufs = bufs_and_sems[:_NBUF]
    gsems = bufs_and_sems[_NBUF]
    ssems = bufs_and_sems[_NBUF + 1]
    nchunks = x_ref.shape[0] // _CHUNK_ROWS

    def gather(i):
        b = i % _NBUF
        return pltpu.make_async_copy(
            x_ref.at[pl.ds(i * _CHUNK_ROWS, _CHUNK_ROWS)], bufs[b], gsems.at[b])

    def scatter(i):
        b = i % _NBUF
        return pltpu.make_async_copy(
            bufs[b], o_ref.at[pl.ds(i * _CHUNK_ROWS, _CHUNK_ROWS)], ssems.at[b])

    for i in range(_NBUF - 1):
        gather(i).start()
    for i in range(nchunks):
        gather(i).wait()
        scatter(i).start()
        nxt = i + _NBUF - 1
        if nxt < nchunks:
            if nxt - _NBUF >= 0:
                scatter(nxt - _NBUF).wait()
            gather(nxt).start()
    for i in range(max(0, nchunks - _NBUF), nchunks):
        scatter(i).wait()


def kernel(images_batch, masks_batch):
    del masks_batch
    B, ve_dim, feature_dim = images_batch.shape
    rows = B * ve_dim
    flat = images_batch.reshape(rows, feature_dim)
    out = pl.pallas_call(
        _copy_body,
        in_specs=[pl.BlockSpec(memory_space=pl.ANY)],
        out_specs=pl.BlockSpec(memory_space=pl.ANY),
        scratch_shapes=(
            [pltpu.VMEM((_CHUNK_ROWS, feature_dim), jnp.float32)
             for _ in range(_NBUF)]
            + [pltpu.SemaphoreType.DMA((_NBUF,)),
               pltpu.SemaphoreType.DMA((_NBUF,))]
        ),
        out_shape=jax.ShapeDtypeStruct((rows, feature_dim), flat.dtype),
    )(flat)
    return out.reshape(B, ve_dim, feature_dim)
